# Initial kernel scaffold; baseline (speedup 1.0000x reference)
import jax
import jax.numpy as jnp
from jax.experimental import pallas as pl


def kernel(hidden_states, weight):
    raise NotImplementedError("write your pallas kernel here")


# speedup vs baseline: 1.7284x; 1.7284x over previous
import functools

import jax
import jax.numpy as jnp
from jax.experimental import pallas as pl

NUM_EXPERTS = 64
TOP_K = 8
N_GROUP = 8
TOPK_GROUP = 4
GROUP_SIZE = NUM_EXPERTS // N_GROUP

_NEG_INF = float("-inf")


def _gate_kernel(hs_ref, wt_ref, w_out_ref, i_out_ref):
    hs = hs_ref[...]
    wt = wt_ref[...]
    logits = jax.lax.dot_general(
        wt, hs,
        dimension_numbers=(((1,), (1,)), ((), ())),
        preferred_element_type=jnp.float32,
    )
    scores = jax.nn.sigmoid(logits)
    t = scores.shape[1]

    riota_g = jax.lax.broadcasted_iota(jnp.int32, (GROUP_SIZE, t), 0)
    group_scores = []
    for g in range(N_GROUP):
        blk = scores[g * GROUP_SIZE:(g + 1) * GROUP_SIZE, :]
        m1 = jnp.max(blk, axis=0, keepdims=True)
        r1 = jnp.min(jnp.where(blk == m1, riota_g, GROUP_SIZE),
                     axis=0, keepdims=True)
        m2 = jnp.max(jnp.where(riota_g == r1, _NEG_INF, blk),
                     axis=0, keepdims=True)
        group_scores.append(m1 + m2)
    gs = jnp.concatenate(group_scores, axis=0)

    riota_ng = jax.lax.broadcasted_iota(jnp.int32, (N_GROUP, t), 0)
    beats = jnp.zeros((N_GROUP, t), dtype=jnp.int32)
    for h in range(N_GROUP):
        gh = gs[h:h + 1, :]
        beats += ((gh > gs) | ((gh == gs) & (h < riota_ng))).astype(jnp.int32)
    sel = beats < TOPK_GROUP

    masked_rows = []
    for g in range(N_GROUP):
        blk = scores[g * GROUP_SIZE:(g + 1) * GROUP_SIZE, :]
        masked_rows.append(jnp.where(sel[g:g + 1, :], blk, _NEG_INF))
    x = jnp.concatenate(masked_rows, axis=0)

    riota_e = jax.lax.broadcasted_iota(jnp.int32, (NUM_EXPERTS, t), 0)
    vals, idxs = [], []
    for _ in range(TOP_K):
        m = jnp.max(x, axis=0, keepdims=True)
        r = jnp.min(jnp.where(x == m, riota_e, NUM_EXPERTS),
                    axis=0, keepdims=True)
        vals.append(m)
        idxs.append(r)
        x = jnp.where(riota_e == r, _NEG_INF, x)
    v = jnp.concatenate(vals, axis=0)
    ridx = jnp.concatenate(idxs, axis=0)
    v = v / (jnp.sum(v, axis=0, keepdims=True) + 1e-20)

    w_out_ref[...] = v.T
    i_out_ref[...] = ridx.T


@functools.partial(jax.jit, static_argnames=())
def kernel(hidden_states, weight):
    hs = hidden_states.reshape(-1, hidden_states.shape[-1])
    num_tokens, hidden = hs.shape
    tile = 256
    grid = num_tokens // tile

    w_out, i_out = pl.pallas_call(
        _gate_kernel,
        grid=(grid,),
        in_specs=[
            pl.BlockSpec((tile, hidden), lambda i: (i, 0)),
            pl.BlockSpec((NUM_EXPERTS, hidden), lambda i: (0, 0)),
        ],
        out_specs=[
            pl.BlockSpec((tile, TOP_K), lambda i: (i, 0)),
            pl.BlockSpec((tile, TOP_K), lambda i: (i, 0)),
        ],
        out_shape=[
            jax.ShapeDtypeStruct((num_tokens, TOP_K), jnp.float32),
            jax.ShapeDtypeStruct((num_tokens, TOP_K), jnp.int32),
        ],
    )(hs, weight)
    return w_out, i_out


# speedup vs baseline: 2.0293x; 1.1741x over previous
import functools

import jax
import jax.numpy as jnp
from jax.experimental import pallas as pl

NUM_EXPERTS = 64
TOP_K = 8
N_GROUP = 8
TOPK_GROUP = 4
GROUP_SIZE = NUM_EXPERTS // N_GROUP

_NEG_INF = float("-inf")


def _gate_kernel(hs_ref, wt_ref, w_out_ref, i_out_ref):
    hs = hs_ref[...]
    wt = wt_ref[...]
    logits = jax.lax.dot_general(
        wt, hs,
        dimension_numbers=(((1,), (1,)), ((), ())),
        preferred_element_type=jnp.float32,
    )
    scores = jax.nn.sigmoid(logits)
    t = scores.shape[1]

    riota_g = jax.lax.broadcasted_iota(jnp.int32, (GROUP_SIZE, t), 0)
    group_scores = []
    for g in range(N_GROUP):
        blk = scores[g * GROUP_SIZE:(g + 1) * GROUP_SIZE, :]
        m1 = jnp.max(blk, axis=0, keepdims=True)
        r1 = jnp.min(jnp.where(blk == m1, riota_g, GROUP_SIZE),
                     axis=0, keepdims=True)
        m2 = jnp.max(jnp.where(riota_g == r1, _NEG_INF, blk),
                     axis=0, keepdims=True)
        group_scores.append(m1 + m2)
    gs = jnp.concatenate(group_scores, axis=0)

    riota_ng = jax.lax.broadcasted_iota(jnp.int32, (N_GROUP, t), 0)
    beats = jnp.zeros((N_GROUP, t), dtype=jnp.int32)
    for h in range(N_GROUP):
        gh = gs[h:h + 1, :]
        beats += ((gh > gs) | ((gh == gs) & (h < riota_ng))).astype(jnp.int32)
    sel = beats < TOPK_GROUP

    masked_rows = []
    for g in range(N_GROUP):
        blk = scores[g * GROUP_SIZE:(g + 1) * GROUP_SIZE, :]
        masked_rows.append(jnp.where(sel[g:g + 1, :], blk, _NEG_INF))
    x = jnp.concatenate(masked_rows, axis=0)

    riota_e = jax.lax.broadcasted_iota(jnp.int32, (NUM_EXPERTS, t), 0)
    vals, idxs = [], []
    for _ in range(TOP_K):
        m = jnp.max(x, axis=0, keepdims=True)
        r = jnp.min(jnp.where(x == m, riota_e, NUM_EXPERTS),
                    axis=0, keepdims=True)
        vals.append(m)
        idxs.append(r)
        x = jnp.where(riota_e == r, _NEG_INF, x)
    v = jnp.concatenate(vals, axis=0)
    ridx = jnp.concatenate(idxs, axis=0)
    v = v / (jnp.sum(v, axis=0, keepdims=True) + 1e-20)

    w_out_ref[...] = v.T
    i_out_ref[...] = ridx.T


@functools.partial(jax.jit, static_argnames=())
def kernel(hidden_states, weight):
    hs = hidden_states.reshape(-1, hidden_states.shape[-1])
    num_tokens, hidden = hs.shape
    tile = 512
    grid = num_tokens // tile

    w_out, i_out = pl.pallas_call(
        _gate_kernel,
        grid=(grid,),
        in_specs=[
            pl.BlockSpec((tile, hidden), lambda i: (i, 0)),
            pl.BlockSpec((NUM_EXPERTS, hidden), lambda i: (0, 0)),
        ],
        out_specs=[
            pl.BlockSpec((tile, TOP_K), lambda i: (i, 0)),
            pl.BlockSpec((tile, TOP_K), lambda i: (i, 0)),
        ],
        out_shape=[
            jax.ShapeDtypeStruct((num_tokens, TOP_K), jnp.float32),
            jax.ShapeDtypeStruct((num_tokens, TOP_K), jnp.int32),
        ],
    )(hs, weight)
    return w_out, i_out


# speedup vs baseline: 2.2820x; 1.1245x over previous
import functools

import jax
import jax.numpy as jnp
from jax.experimental import pallas as pl

NUM_EXPERTS = 64
TOP_K = 8
N_GROUP = 8
TOPK_GROUP = 4
GROUP_SIZE = NUM_EXPERTS // N_GROUP

_NEG_INF = float("-inf")


def _gate_kernel(hs_ref, wt_ref, w_out_ref, i_out_ref):
    hs = hs_ref[...]
    wt = wt_ref[...]
    logits = jax.lax.dot_general(
        wt, hs,
        dimension_numbers=(((1,), (1,)), ((), ())),
        preferred_element_type=jnp.float32,
    )
    scores = jax.nn.sigmoid(logits)
    t = scores.shape[1]

    riota_g = jax.lax.broadcasted_iota(jnp.int32, (GROUP_SIZE, t), 0)
    group_scores = []
    for g in range(N_GROUP):
        blk = scores[g * GROUP_SIZE:(g + 1) * GROUP_SIZE, :]
        m1 = jnp.max(blk, axis=0, keepdims=True)
        r1 = jnp.min(jnp.where(blk == m1, riota_g, GROUP_SIZE),
                     axis=0, keepdims=True)
        m2 = jnp.max(jnp.where(riota_g == r1, _NEG_INF, blk),
                     axis=0, keepdims=True)
        group_scores.append(m1 + m2)
    gs = jnp.concatenate(group_scores, axis=0)

    riota_ng = jax.lax.broadcasted_iota(jnp.int32, (N_GROUP, t), 0)
    beats = jnp.zeros((N_GROUP, t), dtype=jnp.int32)
    for h in range(N_GROUP):
        gh = gs[h:h + 1, :]
        beats += ((gh > gs) | ((gh == gs) & (h < riota_ng))).astype(jnp.int32)
    sel = beats < TOPK_GROUP

    masked_rows = []
    for g in range(N_GROUP):
        blk = scores[g * GROUP_SIZE:(g + 1) * GROUP_SIZE, :]
        masked_rows.append(jnp.where(sel[g:g + 1, :], blk, _NEG_INF))
    x = jnp.concatenate(masked_rows, axis=0)

    riota_e = jax.lax.broadcasted_iota(jnp.int32, (NUM_EXPERTS, t), 0)
    vals, idxs = [], []
    for _ in range(TOP_K):
        m = jnp.max(x, axis=0, keepdims=True)
        r = jnp.min(jnp.where(x == m, riota_e, NUM_EXPERTS),
                    axis=0, keepdims=True)
        vals.append(m)
        idxs.append(r)
        x = jnp.where(riota_e == r, _NEG_INF, x)
    v = jnp.concatenate(vals, axis=0)
    ridx = jnp.concatenate(idxs, axis=0)
    v = v / (jnp.sum(v, axis=0, keepdims=True) + 1e-20)

    w_out_ref[...] = v.T
    i_out_ref[...] = ridx.T


@functools.partial(jax.jit, static_argnames=())
def kernel(hidden_states, weight):
    hs = hidden_states.reshape(-1, hidden_states.shape[-1])
    num_tokens, hidden = hs.shape
    tile = 1024
    grid = num_tokens // tile

    w_out, i_out = pl.pallas_call(
        _gate_kernel,
        grid=(grid,),
        in_specs=[
            pl.BlockSpec((tile, hidden), lambda i: (i, 0)),
            pl.BlockSpec((NUM_EXPERTS, hidden), lambda i: (0, 0)),
        ],
        out_specs=[
            pl.BlockSpec((tile, TOP_K), lambda i: (i, 0)),
            pl.BlockSpec((tile, TOP_K), lambda i: (i, 0)),
        ],
        out_shape=[
            jax.ShapeDtypeStruct((num_tokens, TOP_K), jnp.float32),
            jax.ShapeDtypeStruct((num_tokens, TOP_K), jnp.int32),
        ],
    )(hs, weight)
    return w_out, i_out


# speedup vs baseline: 2.7100x; 1.1875x over previous
import functools

import jax
import jax.numpy as jnp
from jax.experimental import pallas as pl
from jax.experimental.pallas import tpu as pltpu

NUM_EXPERTS = 64
TOP_K = 8
N_GROUP = 8
TOPK_GROUP = 4
GROUP_SIZE = NUM_EXPERTS // N_GROUP

_NEG_INF = float("-inf")


def _gate_kernel(hs_ref, wt_ref, w_out_ref, i_out_ref):
    hs = hs_ref[...]
    wt = wt_ref[...]
    logits = jax.lax.dot_general(
        wt, hs,
        dimension_numbers=(((1,), (1,)), ((), ())),
        preferred_element_type=jnp.float32,
    )
    scores = jax.nn.sigmoid(logits)
    t = scores.shape[1]

    riota_g = jax.lax.broadcasted_iota(jnp.int32, (GROUP_SIZE, t), 0)
    group_scores = []
    for g in range(N_GROUP):
        blk = scores[g * GROUP_SIZE:(g + 1) * GROUP_SIZE, :]
        m1 = jnp.max(blk, axis=0, keepdims=True)
        r1 = jnp.min(jnp.where(blk == m1, riota_g, GROUP_SIZE),
                     axis=0, keepdims=True)
        m2 = jnp.max(jnp.where(riota_g == r1, _NEG_INF, blk),
                     axis=0, keepdims=True)
        group_scores.append(m1 + m2)
    gs = jnp.concatenate(group_scores, axis=0)

    riota_ng = jax.lax.broadcasted_iota(jnp.int32, (N_GROUP, t), 0)
    beats = jnp.zeros((N_GROUP, t), dtype=jnp.int32)
    for h in range(N_GROUP):
        gh = gs[h:h + 1, :]
        beats += ((gh > gs) | ((gh == gs) & (h < riota_ng))).astype(jnp.int32)
    sel = beats < TOPK_GROUP

    masked_rows = []
    for g in range(N_GROUP):
        blk = scores[g * GROUP_SIZE:(g + 1) * GROUP_SIZE, :]
        masked_rows.append(jnp.where(sel[g:g + 1, :], blk, _NEG_INF))
    x = jnp.concatenate(masked_rows, axis=0)

    riota_e = jax.lax.broadcasted_iota(jnp.int32, (NUM_EXPERTS, t), 0)
    vals, idxs = [], []
    for _ in range(TOP_K):
        m = jnp.max(x, axis=0, keepdims=True)
        r = jnp.min(jnp.where(x == m, riota_e, NUM_EXPERTS),
                    axis=0, keepdims=True)
        vals.append(m)
        idxs.append(r)
        x = jnp.where(riota_e == r, _NEG_INF, x)
    v = jnp.concatenate(vals, axis=0)
    ridx = jnp.concatenate(idxs, axis=0)
    v = v / (jnp.sum(v, axis=0, keepdims=True) + 1e-20)

    w_out_ref[...] = v
    i_out_ref[...] = ridx


@functools.partial(jax.jit, static_argnames=())
def kernel(hidden_states, weight):
    hs = hidden_states.reshape(-1, hidden_states.shape[-1])
    num_tokens, hidden = hs.shape
    tile = 1024
    grid = num_tokens // tile

    w_out, i_out = pl.pallas_call(
        _gate_kernel,
        grid=(grid,),
        in_specs=[
            pl.BlockSpec((tile, hidden), lambda i: (i, 0)),
            pl.BlockSpec((NUM_EXPERTS, hidden), lambda i: (0, 0)),
        ],
        out_specs=[
            pl.BlockSpec((TOP_K, tile), lambda i: (0, i)),
            pl.BlockSpec((TOP_K, tile), lambda i: (0, i)),
        ],
        out_shape=[
            jax.ShapeDtypeStruct((TOP_K, num_tokens), jnp.float32),
            jax.ShapeDtypeStruct((TOP_K, num_tokens), jnp.int32),
        ],
        compiler_params=pltpu.CompilerParams(
            dimension_semantics=("arbitrary",),
        ),
    )(hs, weight)
    return w_out.T, i_out.T
